# baseline (speedup 1.0000x reference)
import functools

import jax
import jax.numpy as jnp
from jax import lax
from jax.experimental import pallas as pl
from jax.experimental.pallas import tpu as pltpu
from jax.experimental.pallas import tpu_sc as plsc

N = 50000
E = 800000
F_IN = 128
H = 64
C = 10
G = 64
EPS = 1e-5

CHUNK = 128
TILE_CHUNKS = 400
ROWS2 = 16 * TILE_CHUNKS
E_PAD = ROWS2 * CHUNK
IDX_BLK = 40
SCHUNK = 256
SROWS = E_PAD // SCHUNK
S_TILE_ROWS = SROWS // 16
S_IDX_BLK = 8

ACC_ROWS = 50048
PT = ACC_ROWS // 16

CNT_TILE_CHUNKS = ROWS2 // 32

POOL_TILE_ROWS = 13 * CHUNK
N_POOL = 32 * POOL_TILE_ROWS

RB = 5000
RGRID = N // RB

_MESH = plsc.VectorSubcoreMesh(core_axis_name="c", subcore_axis_name="s")
_f32 = jnp.float32


def _sc_count(dstc, zrows, ones_h):
    @functools.partial(
        pl.kernel,
        out_type=jax.ShapeDtypeStruct((2, ACC_ROWS, 16), _f32),
        mesh=_MESH,
        compiler_params=pltpu.CompilerParams(use_tc_tiling_on_sc=False),
        scratch_types=[
            pltpu.VMEM_SHARED((ACC_ROWS, 16), _f32),
            pltpu.VMEM((CNT_TILE_CHUNKS, CHUNK), jnp.int32),
            pltpu.VMEM((CHUNK, 16), _f32),
        ],
    )
    def k(dst_h, z_h, one_h, out_h, acc, didx, ones_v):
        c = lax.axis_index("c")
        t = lax.axis_index("s")
        pltpu.sync_copy(z_h, acc.at[pl.ds(t * PT, PT)])
        pltpu.sync_copy(one_h, ones_v)
        pltpu.sync_copy(dst_h.at[c * 16 + t], didx)
        plsc.subcore_barrier()

        def inner(j, cc):
            pltpu.sync_copy(ones_v, acc.at[didx.at[j]], add=True)
            return cc

        lax.fori_loop(0, CNT_TILE_CHUNKS, inner, 0)
        plsc.subcore_barrier()
        pltpu.sync_copy(acc.at[pl.ds(t * PT, PT)], out_h.at[c, pl.ds(t * PT, PT)])

    return k(dstc, zrows, ones_h)


def _sc_scatter(src2, dst2, ylo, yhi, zrows):
    @functools.partial(
        pl.kernel,
        out_type=jax.ShapeDtypeStruct((2, ACC_ROWS, 32), _f32),
        mesh=_MESH,
        compiler_params=pltpu.CompilerParams(use_tc_tiling_on_sc=False),
        scratch_types=[
            pltpu.VMEM_SHARED((ACC_ROWS, 32), _f32),
            pltpu.VMEM((S_IDX_BLK, SCHUNK), jnp.int32),
            pltpu.VMEM((S_IDX_BLK, SCHUNK), jnp.int32),
            pltpu.VMEM((2, SCHUNK, 32), _f32),
            pltpu.SemaphoreType.DMA,
            pltpu.SemaphoreType.DMA,
            pltpu.SemaphoreType.DMA,
            pltpu.SemaphoreType.DMA,
        ],
    )
    def k(src_h, dst_h, ylo_h, yhi_h, z_h, out_h, acc, sidx, didx, rows,
          gsem0, gsem1, ssem0, ssem1):
        c = lax.axis_index("c")
        t = lax.axis_index("s")
        pltpu.sync_copy(z_h, acc.at[pl.ds(t * PT, PT)])
        plsc.subcore_barrier()

        NG = S_IDX_BLK

        def run(table):
            def drain(sem):
                pltpu.make_async_copy(table.at[sidx.at[0]], rows.at[0],
                                      sem).wait()

            def blk(b, carry):
                r0 = t * S_TILE_ROWS + b * S_IDX_BLK
                pltpu.sync_copy(src_h.at[pl.ds(r0, S_IDX_BLK)], sidx)
                pltpu.sync_copy(dst_h.at[pl.ds(r0, S_IDX_BLK)], didx)

                pltpu.async_copy(table.at[sidx.at[0]], rows.at[0], gsem0)

                def do_group(g, h, gsem_h, ssem_h, ssem_o, gsem_o):
                    drain(gsem_h)
                    pltpu.async_copy(rows.at[h], acc.at[didx.at[g]],
                                     ssem_h, add=True)

                    @pl.when(g >= 1)
                    def _():
                        drain(ssem_o)

                    @pl.when(g + 1 < NG)
                    def _():
                        pltpu.async_copy(table.at[sidx.at[g + 1]],
                                         rows.at[1 - h], gsem_o)

                def grp_loop(g, cc):
                    @pl.when(lax.rem(g, 2) == 0)
                    def _():
                        do_group(g, 0, gsem0, ssem0, ssem1, gsem1)

                    @pl.when(lax.rem(g, 2) == 1)
                    def _():
                        do_group(g, 1, gsem1, ssem1, ssem0, gsem0)

                    return cc

                lax.fori_loop(0, NG, grp_loop, 0)
                drain(ssem1 if (NG - 1) % 2 == 1 else ssem0)
                return carry

            lax.fori_loop(0, S_TILE_ROWS // S_IDX_BLK, blk, 0)

        @pl.when(c == 0)
        def _():
            run(ylo_h)

        @pl.when(c == 1)
        def _():
            run(yhi_h)

        plsc.subcore_barrier()
        pltpu.sync_copy(acc.at[pl.ds(t * PT, PT)], out_h.at[c, pl.ds(t * PT, PT)])

    return k(src2, dst2, ylo, yhi, zrows)


POOL_ACC_ROWS = G + 16


def _sc_pool(xp, batch2, zsum, zcnt, ones_h):
    @functools.partial(
        pl.kernel,
        out_type=[
            jax.ShapeDtypeStruct((2, G, H), _f32),
            jax.ShapeDtypeStruct((2, G, 16), _f32),
        ],
        mesh=_MESH,
        compiler_params=pltpu.CompilerParams(use_tc_tiling_on_sc=False),
        scratch_types=[
            pltpu.VMEM_SHARED((POOL_ACC_ROWS, H), _f32),
            pltpu.VMEM_SHARED((POOL_ACC_ROWS, 16), _f32),
            pltpu.VMEM((POOL_TILE_ROWS, H), _f32),
            pltpu.VMEM((13, CHUNK), jnp.int32),
            pltpu.VMEM((CHUNK, 16), _f32),
        ],
    )
    def k(x_h, b_h, zs_h, zc_h, one_h, out_h, cnt_h, acc, accc, xv, bidx, ones_v):
        c = lax.axis_index("c")
        t = lax.axis_index("s")

        @pl.when(t == 0)
        def _():
            pltpu.sync_copy(zs_h, acc)
            pltpu.sync_copy(zc_h, accc)

        w = c * 16 + t
        pltpu.sync_copy(x_h.at[pl.ds(w * POOL_TILE_ROWS, POOL_TILE_ROWS)], xv)
        pltpu.sync_copy(b_h.at[w], bidx)
        pltpu.sync_copy(one_h, ones_v)
        plsc.subcore_barrier()

        def inner(j, cc):
            pltpu.sync_copy(xv.at[pl.ds(j * CHUNK, CHUNK)], acc.at[bidx.at[j]],
                            add=True)
            pltpu.sync_copy(ones_v, accc.at[bidx.at[j]], add=True)
            return cc

        lax.fori_loop(0, 13, inner, 0)
        plsc.subcore_barrier()

        @pl.when(t == 0)
        def _():
            pltpu.sync_copy(acc.at[pl.ds(0, G)], out_h.at[c])
            pltpu.sync_copy(accc.at[pl.ds(0, G)], cnt_h.at[c])

    return k(xp, batch2, zsum, zcnt, ones_h)


def _enc_body(cnt_ref, x_ref, w_ref, b_ref, w0_ref,
              x0_ref, dv_ref, ylo_ref, yhi_ref):
    deg = cnt_ref[0, :, 0:1] + cnt_ref[1, :, 0:1] + 1.0
    dv = lax.rsqrt(deg)
    x0 = jnp.dot(x_ref[...], w_ref[...], preferred_element_type=_f32) + b_ref[...]
    y = jnp.dot(x0, w0_ref[...], preferred_element_type=_f32) * dv
    x0_ref[...] = x0
    dv_ref[...] = dv
    ylo_ref[...] = y[:, :32]
    yhi_ref[...] = y[:, 32:]


def _tc_encoder(cnt_p, x, enc_W, enc_b, W0):
    return pl.pallas_call(
        _enc_body,
        grid=(RGRID,),
        in_specs=[
            pl.BlockSpec((2, RB, 16), lambda i: (0, i, 0)),
            pl.BlockSpec((RB, F_IN), lambda i: (i, 0)),
            pl.BlockSpec((F_IN, H), lambda i: (0, 0)),
            pl.BlockSpec((1, H), lambda i: (0, 0)),
            pl.BlockSpec((H, H), lambda i: (0, 0)),
        ],
        out_specs=[
            pl.BlockSpec((RB, H), lambda i: (i, 0)),
            pl.BlockSpec((RB, 1), lambda i: (i, 0)),
            pl.BlockSpec((RB, 32), lambda i: (i, 0)),
            pl.BlockSpec((RB, 32), lambda i: (i, 0)),
        ],
        out_shape=[
            jax.ShapeDtypeStruct((N, H), _f32),
            jax.ShapeDtypeStruct((N, 1), _f32),
            jax.ShapeDtypeStruct((N, 32), _f32),
            jax.ShapeDtypeStruct((N, 32), _f32),
        ],
    )(cnt_p, x, enc_W, enc_b, W0)


def _stats_body(s_ref, ylo_ref, yhi_ref, dv_ref, b_ref, z_ref, st_ref, acc):
    zlo = s_ref[0] + ylo_ref[...]
    zhi = s_ref[1] + yhi_ref[...]
    z = jnp.concatenate([zlo, zhi], axis=1) * dv_ref[...] + b_ref[...]
    z_ref[...] = z
    part = jnp.concatenate(
        [jnp.sum(z, axis=0, keepdims=True),
         jnp.sum(z * z, axis=0, keepdims=True)], axis=1)
    i = pl.program_id(0)

    @pl.when(i == 0)
    def _():
        acc[...] = part

    @pl.when(i > 0)
    def _():
        acc[...] = acc[...] + part

    @pl.when(i == RGRID - 1)
    def _():
        st_ref[...] = acc[...] * (1.0 / N)


def _tc_stats(s, ylo, yhi, dv, b):
    return pl.pallas_call(
        _stats_body,
        grid=(RGRID,),
        in_specs=[
            pl.BlockSpec((2, RB, 32), lambda i: (0, i, 0)),
            pl.BlockSpec((RB, 32), lambda i: (i, 0)),
            pl.BlockSpec((RB, 32), lambda i: (i, 0)),
            pl.BlockSpec((RB, 1), lambda i: (i, 0)),
            pl.BlockSpec((1, H), lambda i: (0, 0)),
        ],
        out_specs=[
            pl.BlockSpec((RB, H), lambda i: (i, 0)),
            pl.BlockSpec((1, 2 * H), lambda i: (0, 0)),
        ],
        out_shape=[
            jax.ShapeDtypeStruct((N, H), _f32),
            jax.ShapeDtypeStruct((1, 2 * H), _f32),
        ],
        scratch_shapes=[pltpu.VMEM((1, 2 * H), _f32)],
    )(s, ylo, yhi, dv, b)


def _norm_body(z_ref, h_ref, st_ref, g_ref, be_ref, dv_ref, wn_ref,
               xn_ref, ylo_ref, yhi_ref):
    mu = st_ref[0:1, 0:H]
    var = st_ref[0:1, H:2 * H] - mu * mu
    xb = (z_ref[...] - mu) * lax.rsqrt(var + EPS) * g_ref[...] + be_ref[...]
    xn = h_ref[...] + jnp.maximum(xb, 0.0)
    xn_ref[...] = xn
    y = jnp.dot(xn, wn_ref[...], preferred_element_type=_f32) * dv_ref[...]
    ylo_ref[...] = y[:, :32]
    yhi_ref[...] = y[:, 32:]


def _tc_norm(z, h, stats, g, be, dv, Wn):
    return pl.pallas_call(
        _norm_body,
        grid=(RGRID,),
        in_specs=[
            pl.BlockSpec((RB, H), lambda i: (i, 0)),
            pl.BlockSpec((RB, H), lambda i: (i, 0)),
            pl.BlockSpec((1, 2 * H), lambda i: (0, 0)),
            pl.BlockSpec((1, H), lambda i: (0, 0)),
            pl.BlockSpec((1, H), lambda i: (0, 0)),
            pl.BlockSpec((RB, 1), lambda i: (i, 0)),
            pl.BlockSpec((H, H), lambda i: (0, 0)),
        ],
        out_specs=[
            pl.BlockSpec((RB, H), lambda i: (i, 0)),
            pl.BlockSpec((RB, 32), lambda i: (i, 0)),
            pl.BlockSpec((RB, 32), lambda i: (i, 0)),
        ],
        out_shape=[
            jax.ShapeDtypeStruct((N, H), _f32),
            jax.ShapeDtypeStruct((N, 32), _f32),
            jax.ShapeDtypeStruct((N, 32), _f32),
        ],
    )(z, h, stats, g, be, dv, Wn)


def _norm_last_body(z_ref, h_ref, st_ref, g_ref, be_ref, xn_ref):
    mu = st_ref[0:1, 0:H]
    var = st_ref[0:1, H:2 * H] - mu * mu
    xb = (z_ref[...] - mu) * lax.rsqrt(var + EPS) * g_ref[...] + be_ref[...]
    xn_ref[...] = h_ref[...] + jnp.maximum(xb, 0.0)


def _tc_norm_last(z, h, stats, g, be):
    return pl.pallas_call(
        _norm_last_body,
        grid=(RGRID,),
        in_specs=[
            pl.BlockSpec((RB, H), lambda i: (i, 0)),
            pl.BlockSpec((RB, H), lambda i: (i, 0)),
            pl.BlockSpec((1, 2 * H), lambda i: (0, 0)),
            pl.BlockSpec((1, H), lambda i: (0, 0)),
            pl.BlockSpec((1, H), lambda i: (0, 0)),
        ],
        out_specs=pl.BlockSpec((RB, H), lambda i: (i, 0)),
        out_shape=jax.ShapeDtypeStruct((N, H), _f32),
    )(z, h, stats, g, be)


def _final_body(sums_ref, cnt_ref, w_ref, lb_ref, out_ref):
    cnt = jnp.maximum(cnt_ref[0, :, 0:1] + cnt_ref[1, :, 0:1], 1.0)
    pooled = (sums_ref[0] + sums_ref[1]) / cnt
    out_ref[...] = (jnp.dot(pooled, w_ref[...], preferred_element_type=_f32)
                    + lb_ref[...])


def _tc_final(sums_p, cnt_p, lin_W, lin_b):
    return pl.pallas_call(
        _final_body,
        grid=(1,),
        in_specs=[
            pl.BlockSpec((2, G, H), lambda i: (0, 0, 0)),
            pl.BlockSpec((2, G, 16), lambda i: (0, 0, 0)),
            pl.BlockSpec((H, C), lambda i: (0, 0)),
            pl.BlockSpec((1, C), lambda i: (0, 0)),
        ],
        out_specs=pl.BlockSpec((G, C), lambda i: (0, 0)),
        out_shape=jax.ShapeDtypeStruct((G, C), _f32),
    )(sums_p, cnt_p, lin_W, lin_b)


def kernel(x, edge_index, batch, enc_W, enc_b,
           conv_W0, conv_b0, bn_g0, bn_b0,
           conv_W1, conv_b1, bn_g1, bn_b1,
           conv_W2, conv_b2, bn_g2, bn_b2,
           lin_W, lin_b):
    pad_e = E_PAD - E
    src2 = jnp.concatenate(
        [edge_index[0], jnp.zeros((pad_e,), jnp.int32)]).reshape(ROWS2, CHUNK)
    dst2 = jnp.concatenate(
        [edge_index[1], jnp.full((pad_e,), N, jnp.int32)]).reshape(ROWS2, CHUNK)
    dstc = dst2.reshape(32, CNT_TILE_CHUNKS, CHUNK)
    z16 = jnp.zeros((PT, 16), _f32)
    z32 = jnp.zeros((PT, 32), _f32)
    ones16 = jnp.ones((CHUNK, 16), _f32)

    cnt_p = _sc_count(dstc, z16, ones16)
    x0, dv, ylo, yhi = _tc_encoder(cnt_p, x, enc_W, enc_b.reshape(1, H), conv_W0)

    h = x0
    next_W = [conv_W1, conv_W2, None]
    bias = [conv_b0, conv_b1, conv_b2]
    gam = [bn_g0, bn_g1, bn_g2]
    bet = [bn_b0, bn_b1, bn_b2]
    for i in range(3):
        s = _sc_scatter(src2.reshape(SROWS, SCHUNK),
                        dst2.reshape(SROWS, SCHUNK), ylo, yhi, z32)
        z, stats = _tc_stats(s, ylo, yhi, dv, bias[i].reshape(1, H))
        if i < 2:
            h, ylo, yhi = _tc_norm(z, h, stats, gam[i].reshape(1, H),
                                   bet[i].reshape(1, H), dv, next_W[i])
        else:
            h = _tc_norm_last(z, h, stats, gam[i].reshape(1, H),
                              bet[i].reshape(1, H))

    xp = jnp.concatenate([h, jnp.zeros((N_POOL - N, H), _f32)], axis=0)
    b2 = jnp.concatenate(
        [batch, jnp.full((N_POOL - N,), G, jnp.int32)]).reshape(32, 13, CHUNK)
    zsum = jnp.zeros((POOL_ACC_ROWS, H), _f32)
    zcnt = jnp.zeros((POOL_ACC_ROWS, 16), _f32)
    sums_p, cnt_pool = _sc_pool(xp, b2, zsum, zcnt, ones16)
    return _tc_final(sums_p, cnt_pool, lin_W, lin_b.reshape(1, C))


# speedup vs baseline: 1.0052x; 1.0052x over previous
import functools

import jax
import jax.numpy as jnp
from jax import lax
from jax.experimental import pallas as pl
from jax.experimental.pallas import tpu as pltpu
from jax.experimental.pallas import tpu_sc as plsc

N = 50000
E = 800000
F_IN = 128
H = 64
C = 10
G = 64
EPS = 1e-5

CHUNK = 128
TILE_CHUNKS = 400
ROWS2 = 16 * TILE_CHUNKS
E_PAD = ROWS2 * CHUNK
IDX_BLK = 40
SCHUNK = 64
SROWS = E_PAD // SCHUNK
S_TILE_ROWS = SROWS // 16
S_IDX_BLK = 40
SGRP = 4

ACC_ROWS = 50048
PT = ACC_ROWS // 16

CNT_TILE_CHUNKS = ROWS2 // 32

POOL_TILE_ROWS = 13 * CHUNK
N_POOL = 32 * POOL_TILE_ROWS

RB = 5000
RGRID = N // RB

_MESH = plsc.VectorSubcoreMesh(core_axis_name="c", subcore_axis_name="s")
_f32 = jnp.float32


def _sc_count(dstc, zrows, ones_h):
    @functools.partial(
        pl.kernel,
        out_type=jax.ShapeDtypeStruct((2, ACC_ROWS, 16), _f32),
        mesh=_MESH,
        compiler_params=pltpu.CompilerParams(use_tc_tiling_on_sc=False),
        scratch_types=[
            pltpu.VMEM_SHARED((ACC_ROWS, 16), _f32),
            pltpu.VMEM((CNT_TILE_CHUNKS, CHUNK), jnp.int32),
            pltpu.VMEM((CHUNK, 16), _f32),
        ],
    )
    def k(dst_h, z_h, one_h, out_h, acc, didx, ones_v):
        c = lax.axis_index("c")
        t = lax.axis_index("s")
        pltpu.sync_copy(z_h, acc.at[pl.ds(t * PT, PT)])
        pltpu.sync_copy(one_h, ones_v)
        pltpu.sync_copy(dst_h.at[c * 16 + t], didx)
        plsc.subcore_barrier()

        def inner(j, cc):
            pltpu.sync_copy(ones_v, acc.at[didx.at[j]], add=True)
            return cc

        lax.fori_loop(0, CNT_TILE_CHUNKS, inner, 0)
        plsc.subcore_barrier()
        pltpu.sync_copy(acc.at[pl.ds(t * PT, PT)], out_h.at[c, pl.ds(t * PT, PT)])

    return k(dstc, zrows, ones_h)


def _sc_scatter(src2, dst2, ylo, yhi, zrows):
    @functools.partial(
        pl.kernel,
        out_type=jax.ShapeDtypeStruct((2, ACC_ROWS, 32), _f32),
        mesh=_MESH,
        compiler_params=pltpu.CompilerParams(use_tc_tiling_on_sc=False),
        scratch_types=[
            pltpu.VMEM_SHARED((ACC_ROWS, 32), _f32),
            pltpu.VMEM((S_IDX_BLK, SCHUNK), jnp.int32),
            pltpu.VMEM((S_IDX_BLK, SCHUNK), jnp.int32),
            pltpu.VMEM((2, SGRP, SCHUNK, 32), _f32),
            pltpu.SemaphoreType.DMA,
            pltpu.SemaphoreType.DMA,
            pltpu.SemaphoreType.DMA,
            pltpu.SemaphoreType.DMA,
        ],
    )
    def k(src_h, dst_h, ylo_h, yhi_h, z_h, out_h, acc, sidx, didx, rows,
          gsem0, gsem1, ssem0, ssem1):
        c = lax.axis_index("c")
        t = lax.axis_index("s")
        pltpu.sync_copy(z_h, acc.at[pl.ds(t * PT, PT)])
        plsc.subcore_barrier()

        NG = S_IDX_BLK // SGRP

        def run(table):
            def drain(sem):
                pltpu.make_async_copy(table.at[sidx.at[0]],
                                      rows.at[0, 0], sem).wait()

            def blk(b, carry):
                r0 = t * S_TILE_ROWS + b * S_IDX_BLK
                pltpu.sync_copy(src_h.at[pl.ds(r0, S_IDX_BLK)], sidx)
                pltpu.sync_copy(dst_h.at[pl.ds(r0, S_IDX_BLK)], didx)

                for i in range(SGRP):
                    pltpu.async_copy(table.at[sidx.at[i]], rows.at[0, i],
                                     gsem0)

                def do_group(g, h, gsem_h, ssem_h, ssem_o, gsem_o):
                    base = g * SGRP
                    for i in range(SGRP):
                        drain(gsem_h)
                    for i in range(SGRP):
                        pltpu.async_copy(rows.at[h, i],
                                         acc.at[didx.at[base + i]],
                                         ssem_h, add=True)

                    @pl.when(g >= 1)
                    def _():
                        for i in range(SGRP):
                            drain(ssem_o)

                    @pl.when(g + 1 < NG)
                    def _():
                        for i in range(SGRP):
                            pltpu.async_copy(
                                table.at[sidx.at[base + SGRP + i]],
                                rows.at[1 - h, i], gsem_o)

                def grp_loop(g, cc):
                    @pl.when(lax.rem(g, 2) == 0)
                    def _():
                        do_group(g, 0, gsem0, ssem0, ssem1, gsem1)

                    @pl.when(lax.rem(g, 2) == 1)
                    def _():
                        do_group(g, 1, gsem1, ssem1, ssem0, gsem0)

                    return cc

                lax.fori_loop(0, NG, grp_loop, 0)
                for i in range(SGRP):
                    drain(ssem1 if (NG - 1) % 2 == 1 else ssem0)
                return carry

            lax.fori_loop(0, S_TILE_ROWS // S_IDX_BLK, blk, 0)

        @pl.when(c == 0)
        def _():
            run(ylo_h)

        @pl.when(c == 1)
        def _():
            run(yhi_h)

        plsc.subcore_barrier()
        pltpu.sync_copy(acc.at[pl.ds(t * PT, PT)], out_h.at[c, pl.ds(t * PT, PT)])

    return k(src2, dst2, ylo, yhi, zrows)


POOL_ACC_ROWS = G + 16


def _sc_pool(xp, batch2, zsum, zcnt, ones_h):
    @functools.partial(
        pl.kernel,
        out_type=[
            jax.ShapeDtypeStruct((2, G, H), _f32),
            jax.ShapeDtypeStruct((2, G, 16), _f32),
        ],
        mesh=_MESH,
        compiler_params=pltpu.CompilerParams(use_tc_tiling_on_sc=False),
        scratch_types=[
            pltpu.VMEM_SHARED((POOL_ACC_ROWS, H), _f32),
            pltpu.VMEM_SHARED((POOL_ACC_ROWS, 16), _f32),
            pltpu.VMEM((POOL_TILE_ROWS, H), _f32),
            pltpu.VMEM((13, CHUNK), jnp.int32),
            pltpu.VMEM((CHUNK, 16), _f32),
        ],
    )
    def k(x_h, b_h, zs_h, zc_h, one_h, out_h, cnt_h, acc, accc, xv, bidx, ones_v):
        c = lax.axis_index("c")
        t = lax.axis_index("s")

        @pl.when(t == 0)
        def _():
            pltpu.sync_copy(zs_h, acc)
            pltpu.sync_copy(zc_h, accc)

        w = c * 16 + t
        pltpu.sync_copy(x_h.at[pl.ds(w * POOL_TILE_ROWS, POOL_TILE_ROWS)], xv)
        pltpu.sync_copy(b_h.at[w], bidx)
        pltpu.sync_copy(one_h, ones_v)
        plsc.subcore_barrier()

        def inner(j, cc):
            pltpu.sync_copy(xv.at[pl.ds(j * CHUNK, CHUNK)], acc.at[bidx.at[j]],
                            add=True)
            pltpu.sync_copy(ones_v, accc.at[bidx.at[j]], add=True)
            return cc

        lax.fori_loop(0, 13, inner, 0)
        plsc.subcore_barrier()

        @pl.when(t == 0)
        def _():
            pltpu.sync_copy(acc.at[pl.ds(0, G)], out_h.at[c])
            pltpu.sync_copy(accc.at[pl.ds(0, G)], cnt_h.at[c])

    return k(xp, batch2, zsum, zcnt, ones_h)


def _enc_body(cnt_ref, x_ref, w_ref, b_ref, w0_ref,
              x0_ref, dv_ref, ylo_ref, yhi_ref):
    deg = cnt_ref[0, :, 0:1] + cnt_ref[1, :, 0:1] + 1.0
    dv = lax.rsqrt(deg)
    x0 = jnp.dot(x_ref[...], w_ref[...], preferred_element_type=_f32) + b_ref[...]
    y = jnp.dot(x0, w0_ref[...], preferred_element_type=_f32) * dv
    x0_ref[...] = x0
    dv_ref[...] = dv
    ylo_ref[...] = y[:, :32]
    yhi_ref[...] = y[:, 32:]


def _tc_encoder(cnt_p, x, enc_W, enc_b, W0):
    return pl.pallas_call(
        _enc_body,
        grid=(RGRID,),
        in_specs=[
            pl.BlockSpec((2, RB, 16), lambda i: (0, i, 0)),
            pl.BlockSpec((RB, F_IN), lambda i: (i, 0)),
            pl.BlockSpec((F_IN, H), lambda i: (0, 0)),
            pl.BlockSpec((1, H), lambda i: (0, 0)),
            pl.BlockSpec((H, H), lambda i: (0, 0)),
        ],
        out_specs=[
            pl.BlockSpec((RB, H), lambda i: (i, 0)),
            pl.BlockSpec((RB, 1), lambda i: (i, 0)),
            pl.BlockSpec((RB, 32), lambda i: (i, 0)),
            pl.BlockSpec((RB, 32), lambda i: (i, 0)),
        ],
        out_shape=[
            jax.ShapeDtypeStruct((N, H), _f32),
            jax.ShapeDtypeStruct((N, 1), _f32),
            jax.ShapeDtypeStruct((N, 32), _f32),
            jax.ShapeDtypeStruct((N, 32), _f32),
        ],
    )(cnt_p, x, enc_W, enc_b, W0)


def _stats_body(s_ref, ylo_ref, yhi_ref, dv_ref, b_ref, z_ref, st_ref, acc):
    zlo = s_ref[0] + ylo_ref[...]
    zhi = s_ref[1] + yhi_ref[...]
    z = jnp.concatenate([zlo, zhi], axis=1) * dv_ref[...] + b_ref[...]
    z_ref[...] = z
    part = jnp.concatenate(
        [jnp.sum(z, axis=0, keepdims=True),
         jnp.sum(z * z, axis=0, keepdims=True)], axis=1)
    i = pl.program_id(0)

    @pl.when(i == 0)
    def _():
        acc[...] = part

    @pl.when(i > 0)
    def _():
        acc[...] = acc[...] + part

    @pl.when(i == RGRID - 1)
    def _():
        st_ref[...] = acc[...] * (1.0 / N)


def _tc_stats(s, ylo, yhi, dv, b):
    return pl.pallas_call(
        _stats_body,
        grid=(RGRID,),
        in_specs=[
            pl.BlockSpec((2, RB, 32), lambda i: (0, i, 0)),
            pl.BlockSpec((RB, 32), lambda i: (i, 0)),
            pl.BlockSpec((RB, 32), lambda i: (i, 0)),
            pl.BlockSpec((RB, 1), lambda i: (i, 0)),
            pl.BlockSpec((1, H), lambda i: (0, 0)),
        ],
        out_specs=[
            pl.BlockSpec((RB, H), lambda i: (i, 0)),
            pl.BlockSpec((1, 2 * H), lambda i: (0, 0)),
        ],
        out_shape=[
            jax.ShapeDtypeStruct((N, H), _f32),
            jax.ShapeDtypeStruct((1, 2 * H), _f32),
        ],
        scratch_shapes=[pltpu.VMEM((1, 2 * H), _f32)],
    )(s, ylo, yhi, dv, b)


def _norm_body(z_ref, h_ref, st_ref, g_ref, be_ref, dv_ref, wn_ref,
               xn_ref, ylo_ref, yhi_ref):
    mu = st_ref[0:1, 0:H]
    var = st_ref[0:1, H:2 * H] - mu * mu
    xb = (z_ref[...] - mu) * lax.rsqrt(var + EPS) * g_ref[...] + be_ref[...]
    xn = h_ref[...] + jnp.maximum(xb, 0.0)
    xn_ref[...] = xn
    y = jnp.dot(xn, wn_ref[...], preferred_element_type=_f32) * dv_ref[...]
    ylo_ref[...] = y[:, :32]
    yhi_ref[...] = y[:, 32:]


def _tc_norm(z, h, stats, g, be, dv, Wn):
    return pl.pallas_call(
        _norm_body,
        grid=(RGRID,),
        in_specs=[
            pl.BlockSpec((RB, H), lambda i: (i, 0)),
            pl.BlockSpec((RB, H), lambda i: (i, 0)),
            pl.BlockSpec((1, 2 * H), lambda i: (0, 0)),
            pl.BlockSpec((1, H), lambda i: (0, 0)),
            pl.BlockSpec((1, H), lambda i: (0, 0)),
            pl.BlockSpec((RB, 1), lambda i: (i, 0)),
            pl.BlockSpec((H, H), lambda i: (0, 0)),
        ],
        out_specs=[
            pl.BlockSpec((RB, H), lambda i: (i, 0)),
            pl.BlockSpec((RB, 32), lambda i: (i, 0)),
            pl.BlockSpec((RB, 32), lambda i: (i, 0)),
        ],
        out_shape=[
            jax.ShapeDtypeStruct((N, H), _f32),
            jax.ShapeDtypeStruct((N, 32), _f32),
            jax.ShapeDtypeStruct((N, 32), _f32),
        ],
    )(z, h, stats, g, be, dv, Wn)


def _norm_last_body(z_ref, h_ref, st_ref, g_ref, be_ref, xn_ref):
    mu = st_ref[0:1, 0:H]
    var = st_ref[0:1, H:2 * H] - mu * mu
    xb = (z_ref[...] - mu) * lax.rsqrt(var + EPS) * g_ref[...] + be_ref[...]
    xn_ref[...] = h_ref[...] + jnp.maximum(xb, 0.0)


def _tc_norm_last(z, h, stats, g, be):
    return pl.pallas_call(
        _norm_last_body,
        grid=(RGRID,),
        in_specs=[
            pl.BlockSpec((RB, H), lambda i: (i, 0)),
            pl.BlockSpec((RB, H), lambda i: (i, 0)),
            pl.BlockSpec((1, 2 * H), lambda i: (0, 0)),
            pl.BlockSpec((1, H), lambda i: (0, 0)),
            pl.BlockSpec((1, H), lambda i: (0, 0)),
        ],
        out_specs=pl.BlockSpec((RB, H), lambda i: (i, 0)),
        out_shape=jax.ShapeDtypeStruct((N, H), _f32),
    )(z, h, stats, g, be)


def _final_body(sums_ref, cnt_ref, w_ref, lb_ref, out_ref):
    cnt = jnp.maximum(cnt_ref[0, :, 0:1] + cnt_ref[1, :, 0:1], 1.0)
    pooled = (sums_ref[0] + sums_ref[1]) / cnt
    out_ref[...] = (jnp.dot(pooled, w_ref[...], preferred_element_type=_f32)
                    + lb_ref[...])


def _tc_final(sums_p, cnt_p, lin_W, lin_b):
    return pl.pallas_call(
        _final_body,
        grid=(1,),
        in_specs=[
            pl.BlockSpec((2, G, H), lambda i: (0, 0, 0)),
            pl.BlockSpec((2, G, 16), lambda i: (0, 0, 0)),
            pl.BlockSpec((H, C), lambda i: (0, 0)),
            pl.BlockSpec((1, C), lambda i: (0, 0)),
        ],
        out_specs=pl.BlockSpec((G, C), lambda i: (0, 0)),
        out_shape=jax.ShapeDtypeStruct((G, C), _f32),
    )(sums_p, cnt_p, lin_W, lin_b)


def kernel(x, edge_index, batch, enc_W, enc_b,
           conv_W0, conv_b0, bn_g0, bn_b0,
           conv_W1, conv_b1, bn_g1, bn_b1,
           conv_W2, conv_b2, bn_g2, bn_b2,
           lin_W, lin_b):
    pad_e = E_PAD - E
    src2 = jnp.concatenate(
        [edge_index[0], jnp.zeros((pad_e,), jnp.int32)]).reshape(ROWS2, CHUNK)
    dst2 = jnp.concatenate(
        [edge_index[1], jnp.full((pad_e,), N, jnp.int32)]).reshape(ROWS2, CHUNK)
    dstc = dst2.reshape(32, CNT_TILE_CHUNKS, CHUNK)
    z16 = jnp.zeros((PT, 16), _f32)
    z32 = jnp.zeros((PT, 32), _f32)
    ones16 = jnp.ones((CHUNK, 16), _f32)

    cnt_p = _sc_count(dstc, z16, ones16)
    x0, dv, ylo, yhi = _tc_encoder(cnt_p, x, enc_W, enc_b.reshape(1, H), conv_W0)

    h = x0
    next_W = [conv_W1, conv_W2, None]
    bias = [conv_b0, conv_b1, conv_b2]
    gam = [bn_g0, bn_g1, bn_g2]
    bet = [bn_b0, bn_b1, bn_b2]
    for i in range(3):
        s = _sc_scatter(src2.reshape(SROWS, SCHUNK),
                        dst2.reshape(SROWS, SCHUNK), ylo, yhi, z32)
        z, stats = _tc_stats(s, ylo, yhi, dv, bias[i].reshape(1, H))
        if i < 2:
            h, ylo, yhi = _tc_norm(z, h, stats, gam[i].reshape(1, H),
                                   bet[i].reshape(1, H), dv, next_W[i])
        else:
            h = _tc_norm_last(z, h, stats, gam[i].reshape(1, H),
                              bet[i].reshape(1, H))

    xp = jnp.concatenate([h, jnp.zeros((N_POOL - N, H), _f32)], axis=0)
    b2 = jnp.concatenate(
        [batch, jnp.full((N_POOL - N,), G, jnp.int32)]).reshape(32, 13, CHUNK)
    zsum = jnp.zeros((POOL_ACC_ROWS, H), _f32)
    zcnt = jnp.zeros((POOL_ACC_ROWS, 16), _f32)
    sums_p, cnt_pool = _sc_pool(xp, b2, zsum, zcnt, ones16)
    return _tc_final(sums_p, cnt_pool, lin_W, lin_b.reshape(1, C))


# speedup vs baseline: 1.0730x; 1.0675x over previous
import functools

import jax
import jax.numpy as jnp
from jax import lax
from jax.experimental import pallas as pl
from jax.experimental.pallas import tpu as pltpu
from jax.experimental.pallas import tpu_sc as plsc

N = 50000
E = 800000
F_IN = 128
H = 64
C = 10
G = 64
EPS = 1e-5

CHUNK = 128
TILE_CHUNKS = 400
ROWS2 = 16 * TILE_CHUNKS
E_PAD = ROWS2 * CHUNK
IDX_BLK = 40
NBUF = 4

ACC_ROWS = 50048
PT = ACC_ROWS // 16

CNT_TILE_CHUNKS = ROWS2 // 32

POOL_TILE_ROWS = 13 * CHUNK
N_POOL = 32 * POOL_TILE_ROWS

RB = 5000
RGRID = N // RB

_MESH = plsc.VectorSubcoreMesh(core_axis_name="c", subcore_axis_name="s")
_f32 = jnp.float32


def _sc_count(dstc, zrows, ones_h):
    @functools.partial(
        pl.kernel,
        out_type=jax.ShapeDtypeStruct((2, ACC_ROWS, 16), _f32),
        mesh=_MESH,
        compiler_params=pltpu.CompilerParams(use_tc_tiling_on_sc=False),
        scratch_types=[
            pltpu.VMEM_SHARED((ACC_ROWS, 16), _f32),
            pltpu.VMEM((CNT_TILE_CHUNKS, CHUNK), jnp.int32),
            pltpu.VMEM((CHUNK, 16), _f32),
        ],
    )
    def k(dst_h, z_h, one_h, out_h, acc, didx, ones_v):
        c = lax.axis_index("c")
        t = lax.axis_index("s")
        pltpu.sync_copy(z_h, acc.at[pl.ds(t * PT, PT)])
        pltpu.sync_copy(one_h, ones_v)
        pltpu.sync_copy(dst_h.at[c * 16 + t], didx)
        plsc.subcore_barrier()

        def inner(j, cc):
            pltpu.sync_copy(ones_v, acc.at[didx.at[j]], add=True)
            return cc

        lax.fori_loop(0, CNT_TILE_CHUNKS, inner, 0)
        plsc.subcore_barrier()
        pltpu.sync_copy(acc.at[pl.ds(t * PT, PT)], out_h.at[c, pl.ds(t * PT, PT)])

    return k(dstc, zrows, ones_h)


def _sc_scatter(src2, dst2, ylo, yhi, zrows):
    @functools.partial(
        pl.kernel,
        out_type=jax.ShapeDtypeStruct((2, ACC_ROWS, 32), _f32),
        mesh=_MESH,
        compiler_params=pltpu.CompilerParams(use_tc_tiling_on_sc=False),
        scratch_types=[
            pltpu.VMEM_SHARED((ACC_ROWS, 32), _f32),
            pltpu.VMEM((IDX_BLK, CHUNK), jnp.int32),
            pltpu.VMEM((IDX_BLK, CHUNK), jnp.int32),
            pltpu.VMEM((NBUF, CHUNK, 32), _f32),
            pltpu.SemaphoreType.DMA,
            pltpu.SemaphoreType.DMA,
        ],
    )
    def k(src_h, dst_h, ylo_h, yhi_h, z_h, out_h, acc, sidx, didx, rows,
          gsem, ssem):
        c = lax.axis_index("c")
        t = lax.axis_index("s")
        pltpu.sync_copy(z_h, acc.at[pl.ds(t * PT, PT)])
        plsc.subcore_barrier()

        def run(table):
            def blk(b, carry):
                r0 = t * TILE_CHUNKS + b * IDX_BLK
                pltpu.sync_copy(src_h.at[pl.ds(r0, IDX_BLK)], sidx)
                pltpu.sync_copy(dst_h.at[pl.ds(r0, IDX_BLK)], didx)

                def grp(gi, cc):
                    base = gi * NBUF
                    gets = [
                        pltpu.async_copy(table.at[sidx.at[base + i]],
                                         rows.at[i], gsem)
                        for i in range(NBUF)
                    ]
                    puts = []
                    for i in range(NBUF):
                        gets[i].wait()
                        puts.append(
                            pltpu.async_copy(rows.at[i],
                                             acc.at[didx.at[base + i]],
                                             ssem, add=True))
                    for p in puts:
                        p.wait()
                    return cc

                return lax.fori_loop(0, IDX_BLK // NBUF, grp, carry)

            lax.fori_loop(0, TILE_CHUNKS // IDX_BLK, blk, 0)

        @pl.when(c == 0)
        def _():
            run(ylo_h)

        @pl.when(c == 1)
        def _():
            run(yhi_h)

        plsc.subcore_barrier()
        pltpu.sync_copy(acc.at[pl.ds(t * PT, PT)], out_h.at[c, pl.ds(t * PT, PT)])

    return k(src2, dst2, ylo, yhi, zrows)


POOL_ACC_ROWS = G + 16


def _sc_pool(xp, batch2, zsum, zcnt, ones_h):
    @functools.partial(
        pl.kernel,
        out_type=[
            jax.ShapeDtypeStruct((2, G, H), _f32),
            jax.ShapeDtypeStruct((2, G, 16), _f32),
        ],
        mesh=_MESH,
        compiler_params=pltpu.CompilerParams(use_tc_tiling_on_sc=False),
        scratch_types=[
            pltpu.VMEM_SHARED((POOL_ACC_ROWS, H), _f32),
            pltpu.VMEM_SHARED((POOL_ACC_ROWS, 16), _f32),
            pltpu.VMEM((POOL_TILE_ROWS, H), _f32),
            pltpu.VMEM((13, CHUNK), jnp.int32),
            pltpu.VMEM((CHUNK, 16), _f32),
        ],
    )
    def k(x_h, b_h, zs_h, zc_h, one_h, out_h, cnt_h, acc, accc, xv, bidx, ones_v):
        c = lax.axis_index("c")
        t = lax.axis_index("s")

        @pl.when(t == 0)
        def _():
            pltpu.sync_copy(zs_h, acc)
            pltpu.sync_copy(zc_h, accc)

        w = c * 16 + t
        pltpu.sync_copy(x_h.at[pl.ds(w * POOL_TILE_ROWS, POOL_TILE_ROWS)], xv)
        pltpu.sync_copy(b_h.at[w], bidx)
        pltpu.sync_copy(one_h, ones_v)
        plsc.subcore_barrier()

        def inner(j, cc):
            pltpu.sync_copy(xv.at[pl.ds(j * CHUNK, CHUNK)], acc.at[bidx.at[j]],
                            add=True)
            pltpu.sync_copy(ones_v, accc.at[bidx.at[j]], add=True)
            return cc

        lax.fori_loop(0, 13, inner, 0)
        plsc.subcore_barrier()

        @pl.when(t == 0)
        def _():
            pltpu.sync_copy(acc.at[pl.ds(0, G)], out_h.at[c])
            pltpu.sync_copy(accc.at[pl.ds(0, G)], cnt_h.at[c])

    return k(xp, batch2, zsum, zcnt, ones_h)


def _enc_body(cnt_ref, x_ref, w_ref, b_ref, w0_ref,
              x0_ref, dv_ref, ylo_ref, yhi_ref):
    deg = cnt_ref[0, :, 0:1] + cnt_ref[1, :, 0:1] + 1.0
    dv = lax.rsqrt(deg)
    x0 = jnp.dot(x_ref[...], w_ref[...], preferred_element_type=_f32) + b_ref[...]
    y = jnp.dot(x0, w0_ref[...], preferred_element_type=_f32) * dv
    x0_ref[...] = x0
    dv_ref[...] = dv
    ylo_ref[...] = y[:, :32]
    yhi_ref[...] = y[:, 32:]


def _tc_encoder(cnt_p, x, enc_W, enc_b, W0):
    return pl.pallas_call(
        _enc_body,
        grid=(RGRID,),
        in_specs=[
            pl.BlockSpec((2, RB, 16), lambda i: (0, i, 0)),
            pl.BlockSpec((RB, F_IN), lambda i: (i, 0)),
            pl.BlockSpec((F_IN, H), lambda i: (0, 0)),
            pl.BlockSpec((1, H), lambda i: (0, 0)),
            pl.BlockSpec((H, H), lambda i: (0, 0)),
        ],
        out_specs=[
            pl.BlockSpec((RB, H), lambda i: (i, 0)),
            pl.BlockSpec((RB, 1), lambda i: (i, 0)),
            pl.BlockSpec((RB, 32), lambda i: (i, 0)),
            pl.BlockSpec((RB, 32), lambda i: (i, 0)),
        ],
        out_shape=[
            jax.ShapeDtypeStruct((N, H), _f32),
            jax.ShapeDtypeStruct((N, 1), _f32),
            jax.ShapeDtypeStruct((N, 32), _f32),
            jax.ShapeDtypeStruct((N, 32), _f32),
        ],
    )(cnt_p, x, enc_W, enc_b, W0)


def _stats_body(s_ref, ylo_ref, yhi_ref, dv_ref, b_ref, z_ref, st_ref, acc):
    zlo = s_ref[0] + ylo_ref[...]
    zhi = s_ref[1] + yhi_ref[...]
    z = jnp.concatenate([zlo, zhi], axis=1) * dv_ref[...] + b_ref[...]
    z_ref[...] = z
    part = jnp.concatenate(
        [jnp.sum(z, axis=0, keepdims=True),
         jnp.sum(z * z, axis=0, keepdims=True)], axis=1)
    i = pl.program_id(0)

    @pl.when(i == 0)
    def _():
        acc[...] = part

    @pl.when(i > 0)
    def _():
        acc[...] = acc[...] + part

    @pl.when(i == RGRID - 1)
    def _():
        st_ref[...] = acc[...] * (1.0 / N)


def _tc_stats(s, ylo, yhi, dv, b):
    return pl.pallas_call(
        _stats_body,
        grid=(RGRID,),
        in_specs=[
            pl.BlockSpec((2, RB, 32), lambda i: (0, i, 0)),
            pl.BlockSpec((RB, 32), lambda i: (i, 0)),
            pl.BlockSpec((RB, 32), lambda i: (i, 0)),
            pl.BlockSpec((RB, 1), lambda i: (i, 0)),
            pl.BlockSpec((1, H), lambda i: (0, 0)),
        ],
        out_specs=[
            pl.BlockSpec((RB, H), lambda i: (i, 0)),
            pl.BlockSpec((1, 2 * H), lambda i: (0, 0)),
        ],
        out_shape=[
            jax.ShapeDtypeStruct((N, H), _f32),
            jax.ShapeDtypeStruct((1, 2 * H), _f32),
        ],
        scratch_shapes=[pltpu.VMEM((1, 2 * H), _f32)],
    )(s, ylo, yhi, dv, b)


def _norm_body(z_ref, h_ref, st_ref, g_ref, be_ref, dv_ref, wn_ref,
               xn_ref, ylo_ref, yhi_ref):
    mu = st_ref[0:1, 0:H]
    var = st_ref[0:1, H:2 * H] - mu * mu
    xb = (z_ref[...] - mu) * lax.rsqrt(var + EPS) * g_ref[...] + be_ref[...]
    xn = h_ref[...] + jnp.maximum(xb, 0.0)
    xn_ref[...] = xn
    y = jnp.dot(xn, wn_ref[...], preferred_element_type=_f32) * dv_ref[...]
    ylo_ref[...] = y[:, :32]
    yhi_ref[...] = y[:, 32:]


def _tc_norm(z, h, stats, g, be, dv, Wn):
    return pl.pallas_call(
        _norm_body,
        grid=(RGRID,),
        in_specs=[
            pl.BlockSpec((RB, H), lambda i: (i, 0)),
            pl.BlockSpec((RB, H), lambda i: (i, 0)),
            pl.BlockSpec((1, 2 * H), lambda i: (0, 0)),
            pl.BlockSpec((1, H), lambda i: (0, 0)),
            pl.BlockSpec((1, H), lambda i: (0, 0)),
            pl.BlockSpec((RB, 1), lambda i: (i, 0)),
            pl.BlockSpec((H, H), lambda i: (0, 0)),
        ],
        out_specs=[
            pl.BlockSpec((RB, H), lambda i: (i, 0)),
            pl.BlockSpec((RB, 32), lambda i: (i, 0)),
            pl.BlockSpec((RB, 32), lambda i: (i, 0)),
        ],
        out_shape=[
            jax.ShapeDtypeStruct((N, H), _f32),
            jax.ShapeDtypeStruct((N, 32), _f32),
            jax.ShapeDtypeStruct((N, 32), _f32),
        ],
    )(z, h, stats, g, be, dv, Wn)


def _norm_last_body(z_ref, h_ref, st_ref, g_ref, be_ref, xn_ref):
    mu = st_ref[0:1, 0:H]
    var = st_ref[0:1, H:2 * H] - mu * mu
    xb = (z_ref[...] - mu) * lax.rsqrt(var + EPS) * g_ref[...] + be_ref[...]
    xn_ref[...] = h_ref[...] + jnp.maximum(xb, 0.0)


def _tc_norm_last(z, h, stats, g, be):
    return pl.pallas_call(
        _norm_last_body,
        grid=(RGRID,),
        in_specs=[
            pl.BlockSpec((RB, H), lambda i: (i, 0)),
            pl.BlockSpec((RB, H), lambda i: (i, 0)),
            pl.BlockSpec((1, 2 * H), lambda i: (0, 0)),
            pl.BlockSpec((1, H), lambda i: (0, 0)),
            pl.BlockSpec((1, H), lambda i: (0, 0)),
        ],
        out_specs=pl.BlockSpec((RB, H), lambda i: (i, 0)),
        out_shape=jax.ShapeDtypeStruct((N, H), _f32),
    )(z, h, stats, g, be)


def _final_body(sums_ref, cnt_ref, w_ref, lb_ref, out_ref):
    cnt = jnp.maximum(cnt_ref[0, :, 0:1] + cnt_ref[1, :, 0:1], 1.0)
    pooled = (sums_ref[0] + sums_ref[1]) / cnt
    out_ref[...] = (jnp.dot(pooled, w_ref[...], preferred_element_type=_f32)
                    + lb_ref[...])


def _tc_final(sums_p, cnt_p, lin_W, lin_b):
    return pl.pallas_call(
        _final_body,
        grid=(1,),
        in_specs=[
            pl.BlockSpec((2, G, H), lambda i: (0, 0, 0)),
            pl.BlockSpec((2, G, 16), lambda i: (0, 0, 0)),
            pl.BlockSpec((H, C), lambda i: (0, 0)),
            pl.BlockSpec((1, C), lambda i: (0, 0)),
        ],
        out_specs=pl.BlockSpec((G, C), lambda i: (0, 0)),
        out_shape=jax.ShapeDtypeStruct((G, C), _f32),
    )(sums_p, cnt_p, lin_W, lin_b)


def kernel(x, edge_index, batch, enc_W, enc_b,
           conv_W0, conv_b0, bn_g0, bn_b0,
           conv_W1, conv_b1, bn_g1, bn_b1,
           conv_W2, conv_b2, bn_g2, bn_b2,
           lin_W, lin_b):
    pad_e = E_PAD - E
    src2 = jnp.concatenate(
        [edge_index[0], jnp.zeros((pad_e,), jnp.int32)]).reshape(ROWS2, CHUNK)
    dst2 = jnp.concatenate(
        [edge_index[1], jnp.full((pad_e,), N, jnp.int32)]).reshape(ROWS2, CHUNK)
    dstc = dst2.reshape(32, CNT_TILE_CHUNKS, CHUNK)
    z16 = jnp.zeros((PT, 16), _f32)
    z32 = jnp.zeros((PT, 32), _f32)
    ones16 = jnp.ones((CHUNK, 16), _f32)

    cnt_p = _sc_count(dstc, z16, ones16)
    x0, dv, ylo, yhi = _tc_encoder(cnt_p, x, enc_W, enc_b.reshape(1, H), conv_W0)

    h = x0
    next_W = [conv_W1, conv_W2, None]
    bias = [conv_b0, conv_b1, conv_b2]
    gam = [bn_g0, bn_g1, bn_g2]
    bet = [bn_b0, bn_b1, bn_b2]
    for i in range(3):
        s = _sc_scatter(src2, dst2, ylo, yhi, z32)
        z, stats = _tc_stats(s, ylo, yhi, dv, bias[i].reshape(1, H))
        if i < 2:
            h, ylo, yhi = _tc_norm(z, h, stats, gam[i].reshape(1, H),
                                   bet[i].reshape(1, H), dv, next_W[i])
        else:
            h = _tc_norm_last(z, h, stats, gam[i].reshape(1, H),
                              bet[i].reshape(1, H))

    xp = jnp.concatenate([h, jnp.zeros((N_POOL - N, H), _f32)], axis=0)
    b2 = jnp.concatenate(
        [batch, jnp.full((N_POOL - N,), G, jnp.int32)]).reshape(32, 13, CHUNK)
    zsum = jnp.zeros((POOL_ACC_ROWS, H), _f32)
    zcnt = jnp.zeros((POOL_ACC_ROWS, 16), _f32)
    sums_p, cnt_pool = _sc_pool(xp, b2, zsum, zcnt, ones16)
    return _tc_final(sums_p, cnt_pool, lin_W, lin_b.reshape(1, C))
